# baseline (speedup 1.0000x reference)
import functools

import jax
import jax.numpy as jnp
from jax import lax
from jax.experimental import pallas as pl
from jax.experimental.pallas import tpu as pltpu
from jax.experimental.pallas import tpu_sc as plsc

D = 1536
E = 256
K = 16
H = 384
CAP = 256
T = 2048
EC = E * CAP

NC = 2
NS = 16
NW = NC * NS

_SC_PARAMS = pltpu.CompilerParams(needs_layout_passes=False)
_SC_PARAMS_NT = pltpu.CompilerParams(
    needs_layout_passes=False, use_tc_tiling_on_sc=False
)


TB = 512


def _route_kernel(x_ref, wr_ref, slot_ref, slotg_ref, w_ref, cnt_ref, run_ref):
    pid = pl.program_id(0)

    @pl.when(pid == 0)
    def _():
        run_ref[...] = jnp.zeros_like(run_ref)

    logits = jnp.dot(x_ref[...], wr_ref[...], preferred_element_type=jnp.float32)
    m = jnp.max(logits, axis=1, keepdims=True)
    p = jnp.exp(logits - m)
    p = p / jnp.sum(p, axis=1, keepdims=True)

    lane = lax.broadcasted_iota(jnp.int32, (TB, E), 1)
    ind = jnp.zeros((TB, E), jnp.float32)
    sels = []
    rws = []
    for _ in range(K):
        mx = jnp.max(p, axis=1, keepdims=True)
        idx = jnp.min(jnp.where(p == mx, lane, E), axis=1, keepdims=True)
        oh = lane == idx
        ind = ind + oh.astype(jnp.float32)
        sels.append(idx)
        rws.append(mx)
        p = jnp.where(oh, -1.0, p)
    sel = jnp.concatenate(sels, axis=1)
    rw = jnp.concatenate(rws, axis=1)
    rw = rw / jnp.sum(rw, axis=1, keepdims=True)

    r0 = lax.broadcasted_iota(jnp.int32, (TB, TB), 0)
    c0 = lax.broadcasted_iota(jnp.int32, (TB, TB), 1)
    ltri = (c0 < r0).astype(jnp.float32)
    base = run_ref[0:1, :]
    pcum = jnp.dot(ltri, ind, preferred_element_type=jnp.float32) + base
    run_ref[0:1, :] = base + jnp.sum(ind, axis=0, keepdims=True)

    poss = []
    for k in range(K):
        ohk = lane == sel[:, k : k + 1]
        poss.append(jnp.sum(jnp.where(ohk, pcum, 0.0), axis=1, keepdims=True))
    pos = jnp.concatenate(poss, axis=1).astype(jnp.int32)

    keep = pos < CAP
    slot = jnp.where(keep, sel * CAP + pos, EC)
    slot_ref[...] = slot
    slotg_ref[...] = jnp.minimum(slot, EC - 1)
    w_ref[...] = jnp.where(keep, rw, 0.0)
    cnt_ref[...] = run_ref[0:1, :].astype(jnp.int32)


def _route(x2, wr):
    return pl.pallas_call(
        _route_kernel,
        grid=(T // TB,),
        in_specs=[
            pl.BlockSpec((TB, D), lambda i: (i, 0)),
            pl.BlockSpec((D, E), lambda i: (0, 0)),
        ],
        out_specs=[
            pl.BlockSpec((TB, K), lambda i: (i, 0)),
            pl.BlockSpec((TB, K), lambda i: (i, 0)),
            pl.BlockSpec((TB, K), lambda i: (i, 0)),
            pl.BlockSpec((1, E), lambda i: (0, 0)),
        ],
        out_shape=[
            jax.ShapeDtypeStruct((T, K), jnp.int32),
            jax.ShapeDtypeStruct((T, K), jnp.int32),
            jax.ShapeDtypeStruct((T, K), jnp.float32),
            jax.ShapeDtypeStruct((1, E), jnp.int32),
        ],
        scratch_shapes=[pltpu.VMEM((8, E), jnp.float32)],
    )(x2, wr)


TOK_PER_W = T // NW
SLAB = 8


def _dispatch_body(
    x_hbm, slot_hbm, buf_hbm, slot_v, xv, dup0, dup1, lsem, ssem0, ssem1
):
    wid = lax.axis_index("s") * NC + lax.axis_index("c")
    base = wid * TOK_PER_W
    pltpu.async_copy(
        slot_hbm.at[pl.ds(base * K, TOK_PER_W * K)], slot_v, lsem
    ).wait()

    dups = (dup0, dup1)
    ssems = (ssem0, ssem1)
    scat = {}
    for t in range(TOK_PER_W):
        if t % SLAB == 0:
            pltpu.async_copy(
                x_hbm.at[pl.ds(base + t, SLAB)], xv, lsem
            ).wait()
        tm = t % SLAB
        db = dups[t % 2]
        if t >= 2:
            scat[t - 2].wait()

        def fill(j, _, tm=tm, db=db):
            v = xv[tm, pl.ds(j * 16, 16)]
            for r in range(K):
                db[r, pl.ds(j * 16, 16)] = v
            return 0

        lax.fori_loop(0, D // 16, fill, 0)
        sv = slot_v[pl.ds(t * K, K)]
        scat[t] = pltpu.make_async_copy(db, buf_hbm.at[sv], ssems[t % 2])
        scat[t].start()
    scat[TOK_PER_W - 2].wait()
    scat[TOK_PER_W - 1].wait()


def _dispatch(x2, slot_flat):
    mesh = plsc.VectorSubcoreMesh(core_axis_name="c", subcore_axis_name="s")
    return pl.kernel(
        _dispatch_body,
        out_type=jax.ShapeDtypeStruct((EC + 8, D), jnp.float32),
        mesh=mesh,
        scratch_types=[
            pltpu.VMEM((TOK_PER_W * K,), jnp.int32),
            pltpu.VMEM((SLAB, D), jnp.float32),
            pltpu.VMEM((K, D), jnp.float32),
            pltpu.VMEM((K, D), jnp.float32),
            pltpu.SemaphoreType.DMA,
            pltpu.SemaphoreType.DMA,
            pltpu.SemaphoreType.DMA,
        ],
        compiler_params=_SC_PARAMS,
        name="sc_dispatch",
    )(x2, slot_flat)


EBLK = 64
NRB = CAP // EBLK


def _expert_kernel(cnt_ref, buf_ref, w1_ref, w3_ref, w2_ref, y_ref):
    b = buf_ref[...]
    a = jnp.dot(b, w1_ref[0], preferred_element_type=jnp.float32)
    g = jnp.dot(b, w3_ref[0], preferred_element_type=jnp.float32)
    h = a * jax.nn.sigmoid(a) * g
    y_ref[...] = jnp.dot(h, w2_ref[0], preferred_element_type=jnp.float32)


def _rb_eff(e, rb, cnt):
    nb = lax.clamp(1, (cnt[e] + EBLK - 1) // EBLK, NRB)
    return e * NRB + lax.min(rb, nb - 1)


def _experts(counts, buf, w1, w3, w2):
    grid_spec = pltpu.PrefetchScalarGridSpec(
        num_scalar_prefetch=1,
        grid=(E, NRB),
        in_specs=[
            pl.BlockSpec((EBLK, D), lambda e, rb, cnt: (_rb_eff(e, rb, cnt), 0)),
            pl.BlockSpec((1, D, H), lambda e, rb, cnt: (e, 0, 0)),
            pl.BlockSpec((1, D, H), lambda e, rb, cnt: (e, 0, 0)),
            pl.BlockSpec((1, H, D), lambda e, rb, cnt: (e, 0, 0)),
        ],
        out_specs=pl.BlockSpec(
            (EBLK, D), lambda e, rb, cnt: (_rb_eff(e, rb, cnt), 0)
        ),
    )
    return pl.pallas_call(
        _expert_kernel,
        grid_spec=grid_spec,
        out_shape=jax.ShapeDtypeStruct((EC, D), jnp.float32),
    )(counts, buf, w1, w3, w2)


TOK_PER_W = T // NW


def _combine_body(
    y_hbm, slotg_hbm, w_hbm, out_hbm,
    slot_v, w_v, yv0, yv1, acc_v, lsem, gsem0, gsem1
):
    wid = lax.axis_index("s") * NC + lax.axis_index("c")
    base = wid * TOK_PER_W
    pltpu.async_copy(slotg_hbm.at[pl.ds(base * K, TOK_PER_W * K)], slot_v, lsem).wait()
    pltpu.async_copy(w_hbm.at[pl.ds(base * K, TOK_PER_W * K)], w_v, lsem).wait()

    pltpu.make_async_copy(y_hbm.at[slot_v.at[pl.ds(0, K)]], yv0, gsem0).start()

    def compute_one(t, yv):
        wks = [
            plsc.load_gather(w_v, [jnp.full((16,), K, jnp.int32) * t + k])
            for k in range(K)
        ]

        def chunk_body(j, _):
            acc = jnp.zeros((16,), jnp.float32)
            for k in range(K):
                term = wks[k] * yv[k, pl.ds(j * 16, 16)]
                acc = acc + jnp.where(wks[k] != 0.0, term, 0.0)
            acc_v[pl.ds(j * 16, 16)] = acc
            return 0

        lax.fori_loop(0, D // 16, chunk_body, 0)
        pltpu.sync_copy(acc_v, out_hbm.at[base + t])

    def pair_body(i, _):
        t0 = i * 2
        pltpu.make_async_copy(
            y_hbm.at[slot_v.at[pl.ds(t0 * K + K, K)]], yv1, gsem1
        ).start()
        pltpu.make_async_copy(y_hbm.at[slot_v.at[pl.ds(t0 * K, K)]], yv0, gsem0).wait()
        compute_one(t0, yv0)

        @pl.when(t0 + 2 < TOK_PER_W)
        def _():
            pltpu.make_async_copy(
                y_hbm.at[slot_v.at[pl.ds(t0 * K + 2 * K, K)]], yv0, gsem0
            ).start()

        pltpu.make_async_copy(
            y_hbm.at[slot_v.at[pl.ds(t0 * K + K, K)]], yv1, gsem1
        ).wait()
        compute_one(t0 + 1, yv1)
        return 0

    lax.fori_loop(0, TOK_PER_W // 2, pair_body, 0)


def _combine(y, slotg, w):
    mesh = plsc.VectorSubcoreMesh(core_axis_name="c", subcore_axis_name="s")
    return pl.kernel(
        _combine_body,
        out_type=jax.ShapeDtypeStruct((T, D), jnp.float32),
        mesh=mesh,
        scratch_types=[
            pltpu.VMEM((TOK_PER_W * K,), jnp.int32),
            pltpu.VMEM((TOK_PER_W * K,), jnp.float32),
            pltpu.VMEM((K, D), jnp.float32),
            pltpu.VMEM((K, D), jnp.float32),
            pltpu.VMEM((D,), jnp.float32),
            pltpu.SemaphoreType.DMA,
            pltpu.SemaphoreType.DMA,
            pltpu.SemaphoreType.DMA,
        ],
        compiler_params=_SC_PARAMS,
        name="sc_combine",
    )(y, slotg, w)


@jax.jit
def kernel(x, Wr, w1, w3, w2):
    B, S, _ = x.shape
    x2 = x.reshape(-1, D)
    slot, slotg, w, counts = _route(x2, Wr)
    buf = _dispatch(x2, slot.reshape(-1))
    y = _experts(counts.reshape(-1), buf, w1, w3, w2)
    out = _combine(y, slotg.reshape(-1), w.reshape(-1))
    return out.reshape(B, S, D)


# speedup vs baseline: 1.7139x; 1.7139x over previous
import functools

import jax
import jax.numpy as jnp
from jax import lax
from jax.experimental import pallas as pl
from jax.experimental.pallas import tpu as pltpu
from jax.experimental.pallas import tpu_sc as plsc

D = 1536
E = 256
K = 16
H = 384
CAP = 256
T = 2048
EC = E * CAP

NC = 2
NS = 16
NW = NC * NS

_SC_PARAMS = pltpu.CompilerParams(needs_layout_passes=False)
_SC_PARAMS_NT = pltpu.CompilerParams(
    needs_layout_passes=False, use_tc_tiling_on_sc=False
)


TB = 512


def _route_kernel(x_ref, wr_ref, slot_ref, slotg_ref, w_ref, run_ref):
    pid = pl.program_id(0)

    @pl.when(pid == 0)
    def _():
        run_ref[...] = jnp.zeros_like(run_ref)

    logits = jnp.dot(x_ref[...], wr_ref[...], preferred_element_type=jnp.float32)
    m = jnp.max(logits, axis=1, keepdims=True)
    p = jnp.exp(logits - m)
    p = p / jnp.sum(p, axis=1, keepdims=True)

    lane = lax.broadcasted_iota(jnp.int32, (TB, E), 1)
    ind = jnp.zeros((TB, E), jnp.float32)
    sels = []
    rws = []
    for _ in range(K):
        mx = jnp.max(p, axis=1, keepdims=True)
        idx = jnp.min(jnp.where(p == mx, lane, E), axis=1, keepdims=True)
        oh = lane == idx
        ind = ind + oh.astype(jnp.float32)
        sels.append(idx)
        rws.append(mx)
        p = jnp.where(oh, -1.0, p)
    sel = jnp.concatenate(sels, axis=1)
    rw = jnp.concatenate(rws, axis=1)
    rw = rw / jnp.sum(rw, axis=1, keepdims=True)

    r0 = lax.broadcasted_iota(jnp.int32, (TB, TB), 0)
    c0 = lax.broadcasted_iota(jnp.int32, (TB, TB), 1)
    ltri = (c0 < r0).astype(jnp.float32)
    base = run_ref[0:1, :]
    pcum = jnp.dot(ltri, ind, preferred_element_type=jnp.float32) + base
    run_ref[0:1, :] = base + jnp.sum(ind, axis=0, keepdims=True)

    poss = []
    for k in range(K):
        ohk = lane == sel[:, k : k + 1]
        poss.append(jnp.sum(jnp.where(ohk, pcum, 0.0), axis=1, keepdims=True))
    pos = jnp.concatenate(poss, axis=1).astype(jnp.int32)

    keep = pos < CAP
    slot = jnp.where(keep, sel * CAP + pos, EC)
    slot_ref[...] = slot
    slotg_ref[...] = jnp.minimum(slot, EC - 1)
    w_ref[...] = jnp.where(keep, rw, 0.0)


def _route(x2, wr):
    return pl.pallas_call(
        _route_kernel,
        grid=(T // TB,),
        in_specs=[
            pl.BlockSpec((TB, D), lambda i: (i, 0)),
            pl.BlockSpec((D, E), lambda i: (0, 0)),
        ],
        out_specs=[
            pl.BlockSpec((TB, K), lambda i: (i, 0)),
            pl.BlockSpec((TB, K), lambda i: (i, 0)),
            pl.BlockSpec((TB, K), lambda i: (i, 0)),
        ],
        out_shape=[
            jax.ShapeDtypeStruct((T, K), jnp.int32),
            jax.ShapeDtypeStruct((T, K), jnp.int32),
            jax.ShapeDtypeStruct((T, K), jnp.float32),
        ],
        scratch_shapes=[pltpu.VMEM((8, E), jnp.float32)],
    )(x2, wr)


TOK_PER_W = T // NW
SLAB = 8


def _dispatch_body(
    x_hbm, slot_hbm, buf_hbm, slot_v, xv, dup0, dup1, lsem, ssem0, ssem1
):
    wid = lax.axis_index("s") * NC + lax.axis_index("c")
    base = wid * TOK_PER_W
    pltpu.async_copy(
        slot_hbm.at[pl.ds(base * K, TOK_PER_W * K)], slot_v, lsem
    ).wait()

    dups = (dup0, dup1)
    ssems = (ssem0, ssem1)
    scat = {}
    for t in range(TOK_PER_W):
        if t % SLAB == 0:
            pltpu.async_copy(
                x_hbm.at[pl.ds(base + t, SLAB)], xv, lsem
            ).wait()
        tm = t % SLAB
        db = dups[t % 2]
        if t >= 2:
            scat[t - 2].wait()

        def fill(j, _, tm=tm, db=db):
            v = xv[tm, pl.ds(j * 16, 16)]
            for r in range(K):
                db[r, pl.ds(j * 16, 16)] = v
            return 0

        lax.fori_loop(0, D // 16, fill, 0)
        sv = slot_v[pl.ds(t * K, K)]
        scat[t] = pltpu.make_async_copy(db, buf_hbm.at[sv], ssems[t % 2])
        scat[t].start()
    scat[TOK_PER_W - 2].wait()
    scat[TOK_PER_W - 1].wait()


def _dispatch(x2, slot_flat):
    mesh = plsc.VectorSubcoreMesh(core_axis_name="c", subcore_axis_name="s")
    return pl.kernel(
        _dispatch_body,
        out_type=jax.ShapeDtypeStruct((EC + 8, D), jnp.float32),
        mesh=mesh,
        scratch_types=[
            pltpu.VMEM((TOK_PER_W * K,), jnp.int32),
            pltpu.VMEM((SLAB, D), jnp.float32),
            pltpu.VMEM((K, D), jnp.float32),
            pltpu.VMEM((K, D), jnp.float32),
            pltpu.SemaphoreType.DMA,
            pltpu.SemaphoreType.DMA,
            pltpu.SemaphoreType.DMA,
        ],
        compiler_params=_SC_PARAMS,
        name="sc_dispatch",
    )(x2, slot_flat)


def _expert_kernel(buf_ref, w1_ref, w3_ref, w2_ref, y_ref):
    b = buf_ref[...]
    a = jnp.dot(b, w1_ref[0], preferred_element_type=jnp.float32)
    g = jnp.dot(b, w3_ref[0], preferred_element_type=jnp.float32)
    h = a * jax.nn.sigmoid(a) * g
    y_ref[...] = jnp.dot(h, w2_ref[0], preferred_element_type=jnp.float32)


def _experts(buf, w1, w3, w2):
    return pl.pallas_call(
        _expert_kernel,
        grid=(E,),
        in_specs=[
            pl.BlockSpec((CAP, D), lambda e: (e, 0)),
            pl.BlockSpec((1, D, H), lambda e: (e, 0, 0)),
            pl.BlockSpec((1, D, H), lambda e: (e, 0, 0)),
            pl.BlockSpec((1, H, D), lambda e: (e, 0, 0)),
        ],
        out_specs=pl.BlockSpec((CAP, D), lambda e: (e, 0)),
        out_shape=jax.ShapeDtypeStruct((EC, D), jnp.float32),
    )(buf, w1, w3, w2)


TOK_PER_W = T // NW


def _combine_body(
    y_hbm, slotg_hbm, w_hbm, out_hbm,
    slot_v, w_v, yv0, yv1, yv2, yv3, acc_v,
    lsem, gsem0, gsem1, gsem2, gsem3
):
    wid = lax.axis_index("s") * NC + lax.axis_index("c")
    base = wid * TOK_PER_W
    pltpu.async_copy(slotg_hbm.at[pl.ds(base * K, TOK_PER_W * K)], slot_v, lsem).wait()
    pltpu.async_copy(w_hbm.at[pl.ds(base * K, TOK_PER_W * K)], w_v, lsem).wait()

    yvs = (yv0, yv1, yv2, yv3)
    gsems = (gsem0, gsem1, gsem2, gsem3)

    def gcopy(t, b):
        return pltpu.make_async_copy(
            y_hbm.at[slot_v.at[pl.ds(t * K, K)]], yvs[b], gsems[b]
        )

    gcopy(0, 0).start()
    gcopy(1, 1).start()

    def compute_one(t, yv):
        wks = [
            plsc.load_gather(w_v, [jnp.full((16,), K, jnp.int32) * t + k])
            for k in range(K)
        ]

        def chunk_body(j, _):
            acc = jnp.zeros((16,), jnp.float32)
            for k in range(K):
                term = wks[k] * yv[k, pl.ds(j * 16, 16)]
                acc = acc + jnp.where(wks[k] != 0.0, term, 0.0)
            acc_v[pl.ds(j * 16, 16)] = acc
            return 0

        lax.fori_loop(0, D // 16, chunk_body, 0)
        pltpu.sync_copy(acc_v, out_hbm.at[base + t])

    def quad_body(q, _):
        t0 = q * 4
        gcopy(t0 + 2, 2).start()
        gcopy(t0 + 3, 3).start()
        gcopy(t0, 0).wait()
        compute_one(t0, yv0)

        @pl.when(t0 + 4 < TOK_PER_W)
        def _():
            gcopy(t0 + 4, 0).start()

        gcopy(t0 + 1, 1).wait()
        compute_one(t0 + 1, yv1)

        @pl.when(t0 + 5 < TOK_PER_W)
        def _():
            gcopy(t0 + 5, 1).start()

        gcopy(t0 + 2, 2).wait()
        compute_one(t0 + 2, yv2)
        gcopy(t0 + 3, 3).wait()
        compute_one(t0 + 3, yv3)
        return 0

    lax.fori_loop(0, TOK_PER_W // 4, quad_body, 0)


def _combine(y, slotg, w):
    mesh = plsc.VectorSubcoreMesh(core_axis_name="c", subcore_axis_name="s")
    return pl.kernel(
        _combine_body,
        out_type=jax.ShapeDtypeStruct((T, D), jnp.float32),
        mesh=mesh,
        scratch_types=[
            pltpu.VMEM((TOK_PER_W * K,), jnp.int32),
            pltpu.VMEM((TOK_PER_W * K,), jnp.float32),
            pltpu.VMEM((K, D), jnp.float32),
            pltpu.VMEM((K, D), jnp.float32),
            pltpu.VMEM((K, D), jnp.float32),
            pltpu.VMEM((K, D), jnp.float32),
            pltpu.VMEM((D,), jnp.float32),
            pltpu.SemaphoreType.DMA,
            pltpu.SemaphoreType.DMA,
            pltpu.SemaphoreType.DMA,
            pltpu.SemaphoreType.DMA,
            pltpu.SemaphoreType.DMA,
        ],
        compiler_params=_SC_PARAMS,
        name="sc_combine",
    )(y, slotg, w)


@jax.jit
def kernel(x, Wr, w1, w3, w2):
    B, S, _ = x.shape
    x2 = x.reshape(-1, D)
    slot, slotg, w = _route(x2, Wr)
    buf = _dispatch(x2, slot.reshape(-1))
    y = _experts(buf, w1, w3, w2)
    out = _combine(y, slotg.reshape(-1), w.reshape(-1))
    return out.reshape(B, S, D)


# speedup vs baseline: 1.9881x; 1.1599x over previous
import functools

import jax
import jax.numpy as jnp
from jax import lax
from jax.experimental import pallas as pl
from jax.experimental.pallas import tpu as pltpu
from jax.experimental.pallas import tpu_sc as plsc

D = 1536
E = 256
K = 16
H = 384
CAP = 256
T = 2048
EC = E * CAP
DP = D // 2

NC = 2
NS = 16
NW = NC * NS

_SC_PARAMS = pltpu.CompilerParams(needs_layout_passes=False)
_SC_PARAMS_NT = pltpu.CompilerParams(
    needs_layout_passes=False, use_tc_tiling_on_sc=False
)


TB = 512


def _route_kernel(x_ref, wr_ref, slot_ref, slotg_ref, w_ref, run_ref):
    pid = pl.program_id(0)

    @pl.when(pid == 0)
    def _():
        run_ref[...] = jnp.zeros_like(run_ref)

    logits = jnp.dot(x_ref[...], wr_ref[...], preferred_element_type=jnp.float32)
    m = jnp.max(logits, axis=1, keepdims=True)
    p = jnp.exp(logits - m)
    p = p / jnp.sum(p, axis=1, keepdims=True)

    lane = lax.broadcasted_iota(jnp.int32, (TB, E), 1)
    ind = jnp.zeros((TB, E), jnp.float32)
    sels = []
    rws = []
    for _ in range(K):
        mx = jnp.max(p, axis=1, keepdims=True)
        idx = jnp.min(jnp.where(p == mx, lane, E), axis=1, keepdims=True)
        oh = lane == idx
        ind = ind + oh.astype(jnp.float32)
        sels.append(idx)
        rws.append(mx)
        p = jnp.where(oh, -1.0, p)
    sel = jnp.concatenate(sels, axis=1)
    rw = jnp.concatenate(rws, axis=1)
    rw = rw / jnp.sum(rw, axis=1, keepdims=True)

    r0 = lax.broadcasted_iota(jnp.int32, (TB, TB), 0)
    c0 = lax.broadcasted_iota(jnp.int32, (TB, TB), 1)
    ltri = (c0 < r0).astype(jnp.float32)
    base = run_ref[0:1, :]
    pcum = jnp.dot(ltri, ind, preferred_element_type=jnp.float32) + base
    run_ref[0:1, :] = base + jnp.sum(ind, axis=0, keepdims=True)

    poss = []
    for k in range(K):
        ohk = lane == sel[:, k : k + 1]
        poss.append(jnp.sum(jnp.where(ohk, pcum, 0.0), axis=1, keepdims=True))
    pos = jnp.concatenate(poss, axis=1).astype(jnp.int32)

    keep = pos < CAP
    slot = jnp.where(keep, sel * CAP + pos, EC)
    slot_ref[...] = slot
    slotg_ref[...] = jnp.minimum(slot, EC - 1)
    w_ref[...] = jnp.where(keep, rw, 0.0)


def _route(x2, wr):
    return pl.pallas_call(
        _route_kernel,
        grid=(T // TB,),
        in_specs=[
            pl.BlockSpec((TB, D), lambda i: (i, 0)),
            pl.BlockSpec((D, E), lambda i: (0, 0)),
        ],
        out_specs=[
            pl.BlockSpec((TB, K), lambda i: (i, 0)),
            pl.BlockSpec((TB, K), lambda i: (i, 0)),
            pl.BlockSpec((TB, K), lambda i: (i, 0)),
        ],
        out_shape=[
            jax.ShapeDtypeStruct((T, K), jnp.int32),
            jax.ShapeDtypeStruct((T, K), jnp.int32),
            jax.ShapeDtypeStruct((T, K), jnp.float32),
        ],
        scratch_shapes=[pltpu.VMEM((8, E), jnp.float32)],
    )(x2, wr)


TOK_PER_W = T // NW
SLAB = 8


def _dispatch_body(
    x_hbm, slot_hbm, buf_hbm, slot_v, xv, dup0, dup1, lsem, ssem0, ssem1
):
    wid = lax.axis_index("s") * NC + lax.axis_index("c")
    base = wid * TOK_PER_W
    pltpu.async_copy(
        slot_hbm.at[pl.ds(base * K, TOK_PER_W * K)], slot_v, lsem
    ).wait()

    dups = (dup0, dup1)
    ssems = (ssem0, ssem1)
    scat = {}
    for t in range(TOK_PER_W):
        if t % SLAB == 0:
            pltpu.async_copy(
                x_hbm.at[pl.ds(base + t, SLAB)], xv, lsem
            ).wait()
        tm = t % SLAB
        db = dups[t % 2]
        if t >= 2:
            scat[t - 2].wait()

        def fill(j, _, tm=tm, db=db):
            v = xv[tm, pl.ds(j * 16, 16)]
            for r in range(K):
                db[r, pl.ds(j * 16, 16)] = v
            return 0

        lax.fori_loop(0, DP // 16, fill, 0)
        sv = slot_v[pl.ds(t * K, K)]
        scat[t] = pltpu.make_async_copy(db, buf_hbm.at[sv], ssems[t % 2])
        scat[t].start()
    scat[TOK_PER_W - 2].wait()
    scat[TOK_PER_W - 1].wait()


def _dispatch(x2, slot_flat):
    mesh = plsc.VectorSubcoreMesh(core_axis_name="c", subcore_axis_name="s")
    return pl.kernel(
        _dispatch_body,
        out_type=jax.ShapeDtypeStruct((EC + 8, DP), jnp.int32),
        mesh=mesh,
        scratch_types=[
            pltpu.VMEM((TOK_PER_W * K,), jnp.int32),
            pltpu.VMEM((SLAB, DP), jnp.int32),
            pltpu.VMEM((K, DP), jnp.int32),
            pltpu.VMEM((K, DP), jnp.int32),
            pltpu.SemaphoreType.DMA,
            pltpu.SemaphoreType.DMA,
            pltpu.SemaphoreType.DMA,
        ],
        compiler_params=_SC_PARAMS,
        name="sc_dispatch",
    )(x2, slot_flat)


def _expert_kernel(buf_ref, w1_ref, w3_ref, w2_ref, y_ref):
    wi = buf_ref[...]
    lo = lax.bitcast_convert_type(lax.shift_left(wi, 16), jnp.float32)
    hi = lax.bitcast_convert_type(wi & jnp.int32(-65536), jnp.float32)
    b = jnp.concatenate([lo, hi], axis=1).astype(jnp.bfloat16)
    w1b = w1_ref[0].astype(jnp.bfloat16)
    w3b = w3_ref[0].astype(jnp.bfloat16)
    w2b = w2_ref[0].astype(jnp.bfloat16)
    a = jnp.dot(b, w1b, preferred_element_type=jnp.float32)
    g = jnp.dot(b, w3b, preferred_element_type=jnp.float32)
    h = (a * jax.nn.sigmoid(a) * g).astype(jnp.bfloat16)
    y = jnp.dot(h, w2b, preferred_element_type=jnp.float32)

    def rnd16(v):
        r = lax.bitcast_convert_type(v, jnp.int32)
        inc = lax.shift_right_logical(r, 16) & 1
        return lax.shift_right_logical(r + 32767 + inc, 16)

    ylo = rnd16(y[:, :DP])
    yhi = rnd16(y[:, DP:])
    y_ref[...] = lax.shift_left(yhi, 16) | ylo


def _experts(buf, w1, w3, w2):
    return pl.pallas_call(
        _expert_kernel,
        grid=(E,),
        in_specs=[
            pl.BlockSpec((CAP, DP), lambda e: (e, 0)),
            pl.BlockSpec((1, D, H), lambda e: (e, 0, 0)),
            pl.BlockSpec((1, D, H), lambda e: (e, 0, 0)),
            pl.BlockSpec((1, H, D), lambda e: (e, 0, 0)),
        ],
        out_specs=pl.BlockSpec((CAP, DP), lambda e: (e, 0)),
        out_shape=jax.ShapeDtypeStruct((EC, DP), jnp.int32),
    )(buf, w1, w3, w2)


TOK_PER_W = T // NW


def _combine_body(
    y_hbm, slotg_hbm, w_hbm, out_hbm,
    slot_v, w_v, yv0, yv1, yv2, yv3, acc_v,
    lsem, gsem0, gsem1, gsem2, gsem3
):
    wid = lax.axis_index("s") * NC + lax.axis_index("c")
    base = wid * TOK_PER_W
    pltpu.async_copy(slotg_hbm.at[pl.ds(base * K, TOK_PER_W * K)], slot_v, lsem).wait()
    pltpu.async_copy(w_hbm.at[pl.ds(base * K, TOK_PER_W * K)], w_v, lsem).wait()

    yvs = (yv0, yv1, yv2, yv3)
    gsems = (gsem0, gsem1, gsem2, gsem3)

    def gcopy(t, b):
        return pltpu.make_async_copy(
            y_hbm.at[slot_v.at[pl.ds(t * K, K)]], yvs[b], gsems[b]
        )

    gcopy(0, 0).start()
    gcopy(1, 1).start()

    def compute_one(t, yv):
        wks = [
            plsc.load_gather(w_v, [jnp.full((16,), K, jnp.int32) * t + k])
            for k in range(K)
        ]

        def chunk_body(j, _):
            acc_lo = jnp.zeros((16,), jnp.float32)
            acc_hi = jnp.zeros((16,), jnp.float32)
            for k in range(K):
                wi = yv[k, pl.ds(j * 16, 16)]
                lo = plsc.bitcast(lax.shift_left(wi, 16), jnp.float32)
                hi = plsc.bitcast(wi & jnp.int32(-65536), jnp.float32)
                nz = wks[k] != 0.0
                acc_lo = acc_lo + jnp.where(nz, wks[k] * lo, 0.0)
                acc_hi = acc_hi + jnp.where(nz, wks[k] * hi, 0.0)
            acc_v[pl.ds(j * 16, 16)] = acc_lo
            acc_v[pl.ds(DP + j * 16, 16)] = acc_hi
            return 0

        lax.fori_loop(0, DP // 16, chunk_body, 0)
        pltpu.sync_copy(acc_v, out_hbm.at[base + t])

    def quad_body(q, _):
        t0 = q * 4
        gcopy(t0 + 2, 2).start()
        gcopy(t0 + 3, 3).start()
        gcopy(t0, 0).wait()
        compute_one(t0, yv0)

        @pl.when(t0 + 4 < TOK_PER_W)
        def _():
            gcopy(t0 + 4, 0).start()

        gcopy(t0 + 1, 1).wait()
        compute_one(t0 + 1, yv1)

        @pl.when(t0 + 5 < TOK_PER_W)
        def _():
            gcopy(t0 + 5, 1).start()

        gcopy(t0 + 2, 2).wait()
        compute_one(t0 + 2, yv2)
        gcopy(t0 + 3, 3).wait()
        compute_one(t0 + 3, yv3)
        return 0

    lax.fori_loop(0, TOK_PER_W // 4, quad_body, 0)


def _combine(y, slotg, w):
    mesh = plsc.VectorSubcoreMesh(core_axis_name="c", subcore_axis_name="s")
    return pl.kernel(
        _combine_body,
        out_type=jax.ShapeDtypeStruct((T, D), jnp.float32),
        mesh=mesh,
        scratch_types=[
            pltpu.VMEM((TOK_PER_W * K,), jnp.int32),
            pltpu.VMEM((TOK_PER_W * K,), jnp.float32),
            pltpu.VMEM((K, DP), jnp.int32),
            pltpu.VMEM((K, DP), jnp.int32),
            pltpu.VMEM((K, DP), jnp.int32),
            pltpu.VMEM((K, DP), jnp.int32),
            pltpu.VMEM((D,), jnp.float32),
            pltpu.SemaphoreType.DMA,
            pltpu.SemaphoreType.DMA,
            pltpu.SemaphoreType.DMA,
            pltpu.SemaphoreType.DMA,
            pltpu.SemaphoreType.DMA,
        ],
        compiler_params=_SC_PARAMS,
        name="sc_combine",
    )(y, slotg, w)


@jax.jit
def kernel(x, Wr, w1, w3, w2):
    B, S, _ = x.shape
    x2 = x.reshape(-1, D)
    slot, slotg, w = _route(x2, Wr)
    xb = x2.astype(jnp.bfloat16)
    xi = lax.bitcast_convert_type(
        jnp.stack([xb[:, :DP], xb[:, DP:]], axis=-1), jnp.int32
    )
    buf = _dispatch(xi, slot.reshape(-1))
    y = _experts(buf, w1, w3, w2)
    out = _combine(y, slotg.reshape(-1), w.reshape(-1))
    return out.reshape(B, S, D)
